# baseline (device time: 37418 ns/iter reference)
import functools

import jax
import jax.numpy as jnp
from jax import lax
from jax.experimental import pallas as pl
from jax.experimental.pallas import tpu as pltpu

N_DEV = 4
N_LAYERS = 3


def kernel(x, Win0, Wout0, Win1, Wout1, Win2, Wout2):
    b, d_shard = x.shape
    h_dim = Win0.shape[1]

    def body(x_ref, win0, wout0, win1, wout1, win2, wout2, out_ref,
             src_ref, recv_ref, send_sems, recv_sems):
        my = lax.axis_index("i")

        barrier = pltpu.get_barrier_semaphore()
        for k in range(1, N_DEV):
            peer = lax.rem(my + k, N_DEV)
            pl.semaphore_signal(barrier, inc=1, device_id=(peer,),
                                device_id_type=pl.DeviceIdType.MESH)
        pl.semaphore_wait(barrier, N_DEV - 1)

        wins = [win0, win1, win2]
        wouts = [wout0, wout1, wout2]
        x_cur = x_ref[...].astype(jnp.bfloat16)
        for layer in range(N_LAYERS):
            win = wins[layer][...].astype(jnp.bfloat16)
            partial = jnp.dot(x_cur, win, preferred_element_type=jnp.float32)
            src_ref[layer] = partial.astype(jnp.bfloat16)

            rdmas = []
            for k in range(1, N_DEV):
                peer = lax.rem(my + k, N_DEV)
                rdma = pltpu.make_async_remote_copy(
                    src_ref=src_ref.at[layer],
                    dst_ref=recv_ref.at[layer, k - 1],
                    send_sem=send_sems.at[layer, k - 1],
                    recv_sem=recv_sems.at[layer, k - 1],
                    device_id=(peer,),
                    device_id_type=pl.DeviceIdType.MESH,
                )
                rdma.start()
                rdmas.append(rdma)

            acc = partial
            for k in range(1, N_DEV):
                rdmas[k - 1].wait()
                acc = acc + recv_ref[layer, k - 1].astype(jnp.float32)

            h_act = jnp.maximum(acc, 0.0).astype(jnp.bfloat16)
            wout = wouts[layer][...].astype(jnp.bfloat16)
            res = jnp.dot(h_act, wout, preferred_element_type=jnp.float32)
            if layer < N_LAYERS - 1:
                x_cur = res.astype(jnp.bfloat16)
            else:
                out_ref[...] = res

        @functools.partial(pl.run_scoped, exit_sem=pltpu.SemaphoreType.REGULAR)
        def _(exit_sem):
            for k in range(1, N_DEV):
                peer = lax.rem(my + k, N_DEV)
                pl.semaphore_signal(exit_sem, inc=1, device_id=(peer,),
                                    device_id_type=pl.DeviceIdType.MESH)
            pl.semaphore_wait(exit_sem, N_DEV - 1)

    return pl.pallas_call(
        body,
        out_shape=jax.ShapeDtypeStruct((b, d_shard), jnp.float32),
        in_specs=[pl.BlockSpec(memory_space=pltpu.VMEM)] * 7,
        out_specs=pl.BlockSpec(memory_space=pltpu.VMEM),
        scratch_shapes=[
            pltpu.VMEM((N_LAYERS, b, h_dim), jnp.bfloat16),
            pltpu.VMEM((N_LAYERS, N_DEV - 1, b, h_dim), jnp.bfloat16),
            pltpu.SemaphoreType.DMA((N_LAYERS, N_DEV - 1)),
            pltpu.SemaphoreType.DMA((N_LAYERS, N_DEV - 1)),
        ],
        compiler_params=pltpu.CompilerParams(collective_id=0),
    )(x, Win0, Wout0, Win1, Wout1, Win2, Wout2)


# device time: 35743 ns/iter; 1.0469x vs baseline; 1.0469x over previous
import functools

import jax
import jax.numpy as jnp
from jax import lax
from jax.experimental import pallas as pl
from jax.experimental.pallas import tpu as pltpu

N_DEV = 4
N_LAYERS = 3


def kernel(x, Win0, Wout0, Win1, Wout1, Win2, Wout2):
    b, d_shard = x.shape
    h_dim = Win0.shape[1]
    bq = b // N_DEV

    def body(x_ref, win0, wout0, win1, wout1, win2, wout2, out_ref,
             qsrc, rs_recv, ag_src, ag_recv, xnext, pbuf,
             rs_ssem, rs_rsem, ag_ssem, ag_rsem):
        my = lax.axis_index("i")

        barrier = pltpu.get_barrier_semaphore()
        for k in range(1, N_DEV):
            peer = lax.rem(my + k, N_DEV)
            pl.semaphore_signal(barrier, inc=1, device_id=(peer,),
                                device_id_type=pl.DeviceIdType.MESH)
        pl.semaphore_wait(barrier, N_DEV - 1)

        wins = [win0, win1, win2]
        wouts = [wout0, wout1, wout2]
        x_cur = x_ref[...].astype(jnp.bfloat16)
        for layer in range(N_LAYERS):
            win = wins[layer][...].astype(jnp.bfloat16)
            partial = jnp.dot(x_cur, win, preferred_element_type=jnp.float32)

            pbuf[...] = partial
            for q in range(N_DEV):
                qsrc[layer, q] = partial[q * bq:(q + 1) * bq, :].astype(
                    jnp.bfloat16)
            rs = []
            for k in range(1, N_DEV):
                peer = lax.rem(my + k, N_DEV)
                r = pltpu.make_async_remote_copy(
                    src_ref=qsrc.at[layer, peer],
                    dst_ref=rs_recv.at[layer, k - 1],
                    send_sem=rs_ssem.at[layer, k - 1],
                    recv_sem=rs_rsem.at[layer, k - 1],
                    device_id=(peer,),
                    device_id_type=pl.DeviceIdType.MESH,
                )
                r.start()
                rs.append(r)

            acc = pbuf[pl.ds(my * bq, bq), :]
            for k in range(1, N_DEV):
                rs[k - 1].wait()
                acc = acc + rs_recv[layer, k - 1].astype(jnp.float32)
            relu_q = jnp.maximum(acc, 0.0).astype(jnp.bfloat16)

            ag_src[layer] = relu_q
            ag = []
            for k in range(1, N_DEV):
                peer = lax.rem(my + k, N_DEV)
                r = pltpu.make_async_remote_copy(
                    src_ref=ag_src.at[layer],
                    dst_ref=ag_recv.at[layer, k - 1],
                    send_sem=ag_ssem.at[layer, k - 1],
                    recv_sem=ag_rsem.at[layer, k - 1],
                    device_id=(peer,),
                    device_id_type=pl.DeviceIdType.MESH,
                )
                r.start()
                ag.append(r)

            wout = wouts[layer][...].astype(jnp.bfloat16)
            dst = xnext.at[layer] if layer < N_LAYERS - 1 else out_ref
            res_q = jnp.dot(relu_q, wout,
                            preferred_element_type=jnp.float32)
            if layer < N_LAYERS - 1:
                res_q = res_q.astype(jnp.bfloat16)
            dst[pl.ds(my * bq, bq), :] = res_q
            for k in range(1, N_DEV):
                ag[k - 1].wait()
                src_pos = lax.rem(my - k + N_DEV, N_DEV)
                res_k = jnp.dot(ag_recv[layer, k - 1], wout,
                                preferred_element_type=jnp.float32)
                if layer < N_LAYERS - 1:
                    res_k = res_k.astype(jnp.bfloat16)
                dst[pl.ds(src_pos * bq, bq), :] = res_k
            if layer < N_LAYERS - 1:
                x_cur = xnext[layer]

        @functools.partial(pl.run_scoped, exit_sem=pltpu.SemaphoreType.REGULAR)
        def _(exit_sem):
            for k in range(1, N_DEV):
                peer = lax.rem(my + k, N_DEV)
                pl.semaphore_signal(exit_sem, inc=1, device_id=(peer,),
                                    device_id_type=pl.DeviceIdType.MESH)
            pl.semaphore_wait(exit_sem, N_DEV - 1)

    return pl.pallas_call(
        body,
        out_shape=jax.ShapeDtypeStruct((b, d_shard), jnp.float32),
        in_specs=[pl.BlockSpec(memory_space=pltpu.VMEM)] * 7,
        out_specs=pl.BlockSpec(memory_space=pltpu.VMEM),
        scratch_shapes=[
            pltpu.VMEM((N_LAYERS, N_DEV, bq, h_dim), jnp.bfloat16),
            pltpu.VMEM((N_LAYERS, N_DEV - 1, bq, h_dim), jnp.bfloat16),
            pltpu.VMEM((N_LAYERS, bq, h_dim), jnp.bfloat16),
            pltpu.VMEM((N_LAYERS, N_DEV - 1, bq, h_dim), jnp.bfloat16),
            pltpu.VMEM((N_LAYERS - 1, b, d_shard), jnp.bfloat16),
            pltpu.VMEM((b, h_dim), jnp.float32),
            pltpu.SemaphoreType.DMA((N_LAYERS, N_DEV - 1)),
            pltpu.SemaphoreType.DMA((N_LAYERS, N_DEV - 1)),
            pltpu.SemaphoreType.DMA((N_LAYERS, N_DEV - 1)),
            pltpu.SemaphoreType.DMA((N_LAYERS, N_DEV - 1)),
        ],
        compiler_params=pltpu.CompilerParams(collective_id=0),
    )(x, Win0, Wout0, Win1, Wout1, Win2, Wout2)


# device time: 34653 ns/iter; 1.0798x vs baseline; 1.0315x over previous
import functools

import jax
import jax.numpy as jnp
from jax import lax
from jax.experimental import pallas as pl
from jax.experimental.pallas import tpu as pltpu

N_DEV = 4
N_LAYERS = 3

SEND_ORDER = (2, 1, 3)
WAIT_ORDER = (1, 3, 2)


def kernel(x, Win0, Wout0, Win1, Wout1, Win2, Wout2):
    b, d_shard = x.shape
    h_dim = Win0.shape[1]
    bq = b // N_DEV

    def body(x_ref, win0, wout0, win1, wout1, win2, wout2, out_ref,
             qsrc0, qsrcN, rs_recv, ag_src, ag_recv, pbuf,
             rs_ssem, rs_rsem, ag_ssem, ag_rsem):
        my = lax.axis_index("i")

        barrier = pltpu.get_barrier_semaphore()
        for k in range(1, N_DEV):
            peer = lax.rem(my + k, N_DEV)
            pl.semaphore_signal(barrier, inc=1, device_id=(peer,),
                                device_id_type=pl.DeviceIdType.MESH)
        pl.semaphore_wait(barrier, N_DEV - 1)

        wins = [win0, win1, win2]
        wouts = [wout0, wout1, wout2]

        x_cur = x_ref[...].astype(jnp.bfloat16)
        partial = jnp.dot(x_cur, win0[...].astype(jnp.bfloat16),
                          preferred_element_type=jnp.float32)
        pbuf[...] = partial
        for q in range(N_DEV):
            qsrc0[q] = partial[q * bq:(q + 1) * bq, :].astype(jnp.bfloat16)
        rs0 = [None] * (N_DEV - 1)
        for k in SEND_ORDER:
            peer = lax.rem(my + k, N_DEV)
            r = pltpu.make_async_remote_copy(
                src_ref=qsrc0.at[peer],
                dst_ref=rs_recv.at[0, k - 1],
                send_sem=rs_ssem.at[0, k - 1],
                recv_sem=rs_rsem.at[0, k - 1],
                device_id=(peer,),
                device_id_type=pl.DeviceIdType.MESH,
            )
            r.start()
            rs0[k - 1] = r
        acc = pbuf[pl.ds(my * bq, bq), :]
        for k in WAIT_ORDER:
            rs0[k - 1].wait()
            acc = acc + rs_recv[0, k - 1].astype(jnp.float32)
        relu_q = jnp.maximum(acc, 0.0).astype(jnp.bfloat16)

        ag_src[0] = relu_q
        ag = [None] * (N_DEV - 1)
        for k in SEND_ORDER:
            peer = lax.rem(my + k, N_DEV)
            r = pltpu.make_async_remote_copy(
                src_ref=ag_src.at[0],
                dst_ref=ag_recv.at[0, k - 1],
                send_sem=ag_ssem.at[0, k - 1],
                recv_sem=ag_rsem.at[0, k - 1],
                device_id=(peer,),
                device_id_type=pl.DeviceIdType.MESH,
            )
            r.start()
            ag[k - 1] = r

        for L in range(N_LAYERS):
            last = L == N_LAYERS - 1
            wout = wouts[L][...].astype(jnp.bfloat16)
            if not last:
                win_next = wins[L + 1][...].astype(jnp.bfloat16)

            res_my = jnp.dot(relu_q, wout,
                             preferred_element_type=jnp.float32)
            if last:
                out_ref[pl.ds(my * bq, bq), :] = res_my
            else:
                acc = jnp.dot(res_my.astype(jnp.bfloat16), win_next,
                              preferred_element_type=jnp.float32)

            rs_next = []
            for k in WAIT_ORDER:
                ag[k - 1].wait()
                p = lax.rem(my - k + N_DEV, N_DEV)
                xq = jnp.dot(ag_recv[L, k - 1], wout,
                             preferred_element_type=jnp.float32)
                if last:
                    out_ref[pl.ds(p * bq, bq), :] = xq
                else:
                    p2 = jnp.dot(xq.astype(jnp.bfloat16), win_next,
                                 preferred_element_type=jnp.float32)
                    qsrcN[L, k - 1] = p2.astype(jnp.bfloat16)
                    r = pltpu.make_async_remote_copy(
                        src_ref=qsrcN.at[L, k - 1],
                        dst_ref=rs_recv.at[L + 1, 3 - k],
                        send_sem=rs_ssem.at[L + 1, 3 - k],
                        recv_sem=rs_rsem.at[L + 1, 3 - k],
                        device_id=(p,),
                        device_id_type=pl.DeviceIdType.MESH,
                    )
                    r.start()
                    rs_next.append((3 - k, r))
            if last:
                break

            for slot, r in rs_next:
                r.wait()
                acc = acc + rs_recv[L + 1, slot].astype(jnp.float32)
            relu_q = jnp.maximum(acc, 0.0).astype(jnp.bfloat16)

            ag_src[L + 1] = relu_q
            ag = [None] * (N_DEV - 1)
            for k in SEND_ORDER:
                peer = lax.rem(my + k, N_DEV)
                r = pltpu.make_async_remote_copy(
                    src_ref=ag_src.at[L + 1],
                    dst_ref=ag_recv.at[L + 1, k - 1],
                    send_sem=ag_ssem.at[L + 1, k - 1],
                    recv_sem=ag_rsem.at[L + 1, k - 1],
                    device_id=(peer,),
                    device_id_type=pl.DeviceIdType.MESH,
                )
                r.start()
                ag[k - 1] = r

        @functools.partial(pl.run_scoped, exit_sem=pltpu.SemaphoreType.REGULAR)
        def _(exit_sem):
            for k in range(1, N_DEV):
                peer = lax.rem(my + k, N_DEV)
                pl.semaphore_signal(exit_sem, inc=1, device_id=(peer,),
                                    device_id_type=pl.DeviceIdType.MESH)
            pl.semaphore_wait(exit_sem, N_DEV - 1)

    return pl.pallas_call(
        body,
        out_shape=jax.ShapeDtypeStruct((b, d_shard), jnp.float32),
        in_specs=[pl.BlockSpec(memory_space=pltpu.VMEM)] * 7,
        out_specs=pl.BlockSpec(memory_space=pltpu.VMEM),
        scratch_shapes=[
            pltpu.VMEM((N_DEV, bq, h_dim), jnp.bfloat16),
            pltpu.VMEM((N_LAYERS - 1, N_DEV - 1, bq, h_dim), jnp.bfloat16),
            pltpu.VMEM((N_LAYERS, N_DEV - 1, bq, h_dim), jnp.bfloat16),
            pltpu.VMEM((N_LAYERS, bq, h_dim), jnp.bfloat16),
            pltpu.VMEM((N_LAYERS, N_DEV - 1, bq, h_dim), jnp.bfloat16),
            pltpu.VMEM((b, h_dim), jnp.float32),
            pltpu.SemaphoreType.DMA((N_LAYERS, N_DEV - 1)),
            pltpu.SemaphoreType.DMA((N_LAYERS, N_DEV - 1)),
            pltpu.SemaphoreType.DMA((N_LAYERS, N_DEV - 1)),
            pltpu.SemaphoreType.DMA((N_LAYERS, N_DEV - 1)),
        ],
        compiler_params=pltpu.CompilerParams(collective_id=0),
    )(x, Win0, Wout0, Win1, Wout1, Win2, Wout2)
